# baseline (speedup 1.0000x reference)
import jax
import jax.numpy as jnp
from jax import lax
from jax.experimental import pallas as pl
from jax.experimental.pallas import tpu as pltpu
from jax.experimental.pallas import tpu_sc as plsc

N0, N1, EMB = 384, 384, 96
NC, NS = 2, 16
NW = NC * NS
RPW = N0 // NW
TK, TJ = EMB // 8, N1 // 128
ROW = EMB * N1
QUART = ROW // 4
QTK = TK // 4


def _sc_body(e0_hbm, e1_hbm, out_hbm, e1p, e0v, tmp, bufs, sems):
    wid = lax.axis_index("s") * NC + lax.axis_index("c")
    row0 = wid * RPW

    pltpu.sync_copy(e0_hbm.at[pl.ds(row0 * EMB, RPW * EMB)], e0v)
    i96 = lax.iota(jnp.int32, 16) * EMB

    def _slab(tj, _):
        pltpu.sync_copy(e1_hbm.at[pl.ds(tj * 128 * EMB, 128 * EMB)], tmp)

        @plsc.parallel_loop(0, TK * 8, unroll=2)
        def _run(ts):
            tk = ts // 8
            s = ts - tk * 8
            dst0 = ((tk * TJ + tj) * 8 + s) * 128
            for b in range(8):
                vidx = jnp.full((16,), 8 * tk + s + 1536 * b, jnp.int32) + i96
                e1p[pl.ds(dst0 + 16 * b, 16)] = plsc.load_gather(tmp, [vidx])

        return 0

    lax.fori_loop(0, TJ, _slab, 0)

    def _copy(q, r, h):
        return pltpu.make_async_copy(
            bufs.at[q],
            out_hbm.at[pl.ds((row0 + r) * ROW + h * QUART, QUART)],
            sems.at[q],
        )

    def _row(r, _):
        for h in range(4):

            @pl.when(r >= 1)
            def _():
                _copy(h, r - 1, h).wait()

            @plsc.parallel_loop(0, QTK * 8, unroll=4)
            def _krow(ts):
                t2 = ts // 8
                s = ts - t2 * 8
                tk = h * QTK + t2
                v0 = plsc.load_gather(
                    e0v, [jnp.full((16,), r * EMB + 8 * tk + s, jnp.int32)]
                )
                src0 = ((tk * TJ) * 8 + s) * 128
                dst0 = ((t2 * TJ) * 8 + s) * 128
                for tj in range(TJ):
                    for b in range(8):
                        off = tj * 8 * 128 + 16 * b
                        bufs[h, pl.ds(dst0 + off, 16)] = (
                            v0 + e1p[pl.ds(src0 + off, 16)]
                        )

            _copy(h, r, h).start()
        return 0

    lax.fori_loop(0, RPW, _row, 0)

    for h in range(4):
        _copy(h, RPW - 1, h).wait()


def kernel(x, emb_0, emb_1):
    del x
    e0f = emb_0.reshape(N0 * EMB)
    e1f = emb_1.reshape(N1 * EMB)
    mesh = plsc.VectorSubcoreMesh(core_axis_name="c", subcore_axis_name="s")
    kfn = pl.kernel(
        _sc_body,
        mesh=mesh,
        out_type=jax.ShapeDtypeStruct((N0 * ROW,), jnp.float32),
        scratch_types=[
            pltpu.VMEM((ROW,), jnp.float32),
            pltpu.VMEM((RPW * EMB,), jnp.float32),
            pltpu.VMEM((128 * EMB,), jnp.float32),
            pltpu.VMEM((4, QUART), jnp.float32),
            pltpu.SemaphoreType.DMA((4,)),
        ],
        compiler_params=pltpu.CompilerParams(needs_layout_passes=False),
    )
    out_phy = kfn(e0f, e1f)
    out3 = (
        out_phy.reshape(N0, TK, TJ, 8, 128)
        .transpose(0, 1, 3, 2, 4)
        .reshape(N0, EMB, N1)
    )
    return out3.transpose(0, 2, 1)[None]


# speedup vs baseline: 1.4500x; 1.4500x over previous
import jax
import jax.numpy as jnp
from jax import lax
from jax.experimental import pallas as pl
from jax.experimental.pallas import tpu as pltpu
from jax.experimental.pallas import tpu_sc as plsc

N0, N1, EMB = 384, 384, 96
NC, NS = 2, 16
NW = NC * NS
RPW = N0 // NW
TK, TJ = EMB // 8, N1 // 128
ROW = EMB * N1
HALF = ROW // 2
HTK = TK // 2


def _sc_body(
    e0_hbm, e1_hbm, out_hbm, e1p, e0v, tmp, buf_a, buf_b, buf_c,
    sem_a, sem_b, sem_c,
):
    wid = lax.axis_index("s") * NC + lax.axis_index("c")
    row0 = wid * RPW

    pltpu.sync_copy(e0_hbm.at[pl.ds(row0 * EMB, RPW * EMB)], e0v)
    i96 = lax.iota(jnp.int32, 16) * EMB

    def _slab(tj, _):
        pltpu.sync_copy(e1_hbm.at[pl.ds(tj * 128 * EMB, 128 * EMB)], tmp)

        @plsc.parallel_loop(0, TK * 8, unroll=4)
        def _run(ts):
            tk = ts // 8
            s = ts - tk * 8
            dst0 = ((tk * TJ + tj) * 8 + s) * 128
            for b in range(8):
                vidx = jnp.full((16,), 8 * tk + s + 1536 * b, jnp.int32) + i96
                e1p[pl.ds(dst0 + 16 * b, 16)] = plsc.load_gather(tmp, [vidx])

        return 0

    lax.fori_loop(0, TJ, _slab, 0)

    def _copy(buf, sem, r, h):
        return pltpu.make_async_copy(
            buf,
            out_hbm.at[pl.ds((row0 + r) * ROW + h * HALF, HALF)],
            sem,
        )

    rings = ((buf_a, sem_a), (buf_b, sem_b), (buf_c, sem_c))

    def _group(g, _):
        for j, (buf, sem) in enumerate(rings):
            n = 3 * g + j
            r = n // 2
            h = n - 2 * r

            @pl.when(g >= 1)
            def _():
                rp = (n - 3) // 2
                _copy(buf, sem, rp, (n - 3) - 2 * rp).wait()

            @plsc.parallel_loop(0, HTK * 8, unroll=4)
            def _krow(ts):
                t2 = ts // 8
                s = ts - t2 * 8
                tk = h * HTK + t2
                v0 = plsc.load_gather(
                    e0v, [jnp.full((16,), r * EMB + 8 * tk + s, jnp.int32)]
                )
                src0 = ((tk * TJ) * 8 + s) * 128
                dst0 = ((t2 * TJ) * 8 + s) * 128
                for tj in range(TJ):
                    for b in range(8):
                        off = tj * 8 * 128 + 16 * b
                        buf[pl.ds(dst0 + off, 16)] = (
                            v0 + e1p[pl.ds(src0 + off, 16)]
                        )

            _copy(buf, sem, r, h).start()
        return 0

    lax.fori_loop(0, 2 * RPW // 3, _group, 0)

    for j, (buf, sem) in enumerate(rings):
        n = 2 * RPW - 3 + j
        _copy(buf, sem, n // 2, n % 2).wait()


def kernel(x, emb_0, emb_1):
    del x
    e0f = emb_0.reshape(N0 * EMB)
    e1f = emb_1.reshape(N1 * EMB)
    mesh = plsc.VectorSubcoreMesh(core_axis_name="c", subcore_axis_name="s")
    kfn = pl.kernel(
        _sc_body,
        mesh=mesh,
        out_type=jax.ShapeDtypeStruct((N0 * ROW,), jnp.float32),
        scratch_types=[
            pltpu.VMEM((ROW,), jnp.float32),
            pltpu.VMEM((RPW * EMB,), jnp.float32),
            pltpu.VMEM((128 * EMB,), jnp.float32),
            pltpu.VMEM((HALF,), jnp.float32),
            pltpu.VMEM((HALF,), jnp.float32),
            pltpu.VMEM((HALF,), jnp.float32),
            pltpu.SemaphoreType.DMA,
            pltpu.SemaphoreType.DMA,
            pltpu.SemaphoreType.DMA,
        ],
        compiler_params=pltpu.CompilerParams(needs_layout_passes=False),
    )
    out_phy = kfn(e0f, e1f)
    out3 = (
        out_phy.reshape(N0, TK, TJ, 8, 128)
        .transpose(0, 1, 3, 2, 4)
        .reshape(N0, EMB, N1)
    )
    return out3.transpose(0, 2, 1)[None]


# speedup vs baseline: 1.8216x; 1.2563x over previous
import jax
import jax.numpy as jnp
from jax import lax
from jax.experimental import pallas as pl
from jax.experimental.pallas import tpu as pltpu
from jax.experimental.pallas import tpu_sc as plsc

N0, N1, EMB = 384, 384, 96
NC, NS = 2, 16
NW = NC * NS
RPW = N0 // NW
TK, TJ = EMB // 8, N1 // 128
ROW = EMB * N1
HALF = ROW // 2
HTK = TK // 2


def _sc_body(e0_hbm, e1_hbm, out_hbm, e1p, e0v, buf_a, buf_b, sem_a, sem_b):
    wid = lax.axis_index("s") * NC + lax.axis_index("c")
    row0 = wid * RPW

    pltpu.sync_copy(e0_hbm.at[pl.ds(row0 * EMB, RPW * EMB)], e0v)
    i96 = lax.iota(jnp.int32, 16) * EMB

    slab = 128 * EMB

    def _stage(tj, buf, sem):
        return pltpu.make_async_copy(
            e1_hbm.at[pl.ds(tj * slab, slab)], buf.at[pl.ds(0, slab)], sem
        )

    _stage(0, buf_a, sem_a).start()
    for tj in range(TJ):
        buf, sem = (buf_a, sem_a) if tj % 2 == 0 else (buf_b, sem_b)
        _stage(tj, buf, sem).wait()
        if tj + 1 < TJ:
            nbuf, nsem = (buf_b, sem_b) if tj % 2 == 0 else (buf_a, sem_a)
            _stage(tj + 1, nbuf, nsem).start()

        @plsc.parallel_loop(0, TK * 8, unroll=4)
        def _run(ts):
            tk = ts // 8
            s = ts - tk * 8
            dst0 = ((tk * TJ + tj) * 8 + s) * 128
            for b in range(8):
                vidx = jnp.full((16,), 8 * tk + s + 1536 * b, jnp.int32) + i96
                e1p[pl.ds(dst0 + 16 * b, 16)] = plsc.load_gather(buf, [vidx])

    def _copy(buf, sem, r, h):
        return pltpu.make_async_copy(
            buf,
            out_hbm.at[pl.ds((row0 + r) * ROW + h * HALF, HALF)],
            sem,
        )

    def _row(r, _):
        for h, buf, sem in ((0, buf_a, sem_a), (1, buf_b, sem_b)):

            @pl.when(r >= 1)
            def _():
                _copy(buf, sem, r - 1, h).wait()

            @plsc.parallel_loop(0, HTK * 8, unroll=4)
            def _krow(ts):
                t2 = ts // 8
                s = ts - t2 * 8
                tk = h * HTK + t2
                v0 = plsc.load_gather(
                    e0v, [jnp.full((16,), r * EMB + 8 * tk + s, jnp.int32)]
                )
                src0 = ((tk * TJ) * 8 + s) * 128
                dst0 = ((t2 * TJ) * 8 + s) * 128
                for tj in range(TJ):
                    for b in range(8):
                        off = tj * 8 * 128 + 16 * b
                        buf[pl.ds(dst0 + off, 16)] = (
                            v0 + e1p[pl.ds(src0 + off, 16)]
                        )

            _copy(buf, sem, r, h).start()
        return 0

    lax.fori_loop(0, RPW, _row, 0)

    _copy(buf_a, sem_a, RPW - 1, 0).wait()
    _copy(buf_b, sem_b, RPW - 1, 1).wait()


def kernel(x, emb_0, emb_1):
    del x
    e0f = emb_0.reshape(N0 * EMB)
    e1f = emb_1.reshape(N1 * EMB)
    mesh = plsc.VectorSubcoreMesh(core_axis_name="c", subcore_axis_name="s")
    kfn = pl.kernel(
        _sc_body,
        mesh=mesh,
        out_type=jax.ShapeDtypeStruct((N0 * ROW,), jnp.float32),
        scratch_types=[
            pltpu.VMEM((ROW,), jnp.float32),
            pltpu.VMEM((RPW * EMB,), jnp.float32),
            pltpu.VMEM((HALF,), jnp.float32),
            pltpu.VMEM((HALF,), jnp.float32),
            pltpu.SemaphoreType.DMA,
            pltpu.SemaphoreType.DMA,
        ],
        compiler_params=pltpu.CompilerParams(needs_layout_passes=False),
    )
    out_phy = kfn(e0f, e1f)
    out3 = (
        out_phy.reshape(N0, TK, TJ, 8, 128)
        .transpose(0, 1, 3, 2, 4)
        .reshape(N0, EMB, N1)
    )
    return out3.transpose(0, 2, 1)[None]


# speedup vs baseline: 1.8261x; 1.0025x over previous
import jax
import jax.numpy as jnp
from jax import lax
from jax.experimental import pallas as pl
from jax.experimental.pallas import tpu as pltpu
from jax.experimental.pallas import tpu_sc as plsc

N0, N1, EMB = 384, 384, 96
NC, NS = 2, 16
NW = NC * NS
RPW = N0 // NW
TK, TJ = EMB // 8, N1 // 128
ROW = EMB * N1
HALF = ROW // 2
HTK = TK // 2


def _sc_body(e0_hbm, e1_hbm, out_hbm, e1p, e0v, buf_a, buf_b, sem_a, sem_b):
    wid = lax.axis_index("s") * NC + lax.axis_index("c")
    row0 = wid * RPW

    pltpu.sync_copy(e0_hbm.at[pl.ds(row0 * EMB, RPW * EMB)], e0v)
    i96 = lax.iota(jnp.int32, 16) * EMB

    slab = 128 * EMB

    def _stage(tj, buf, sem):
        return pltpu.make_async_copy(
            e1_hbm.at[pl.ds(tj * slab, slab)], buf.at[pl.ds(0, slab)], sem
        )

    _stage(0, buf_a, sem_a).start()
    for tj in range(TJ):
        buf, sem = (buf_a, sem_a) if tj % 2 == 0 else (buf_b, sem_b)
        _stage(tj, buf, sem).wait()
        if tj + 1 < TJ:
            nbuf, nsem = (buf_b, sem_b) if tj % 2 == 0 else (buf_a, sem_a)
            _stage(tj + 1, nbuf, nsem).start()

        @plsc.parallel_loop(0, TK * 8, unroll=4)
        def _run(ts):
            tk = ts // 8
            s = ts - tk * 8
            dst0 = ((tk * TJ + tj) * 8 + s) * 128
            for b in range(8):
                vidx = jnp.full((16,), 8 * tk + s + 1536 * b, jnp.int32) + i96
                e1p[pl.ds(dst0 + 16 * b, 16)] = plsc.load_gather(buf, [vidx])

    def _copy(buf, sem, r):
        return pltpu.make_async_copy(
            buf,
            out_hbm.at[pl.ds((row0 + r) * ROW, ROW)],
            sem,
        )

    def _pair(p, _):
        for j, buf, sem in ((0, buf_a, sem_a), (1, buf_b, sem_b)):
            r = 2 * p + j

            @pl.when(p >= 1)
            def _():
                _copy(buf, sem, r - 2).wait()

            @plsc.parallel_loop(0, TK * 8, unroll=4)
            def _krow(ts):
                tk = ts // 8
                s = ts - tk * 8
                v0 = plsc.load_gather(
                    e0v, [jnp.full((16,), r * EMB + 8 * tk + s, jnp.int32)]
                )
                dst0 = ((tk * TJ) * 8 + s) * 128
                for tj in range(TJ):
                    for b in range(8):
                        off = tj * 8 * 128 + 16 * b
                        buf[pl.ds(dst0 + off, 16)] = (
                            v0 + e1p[pl.ds(dst0 + off, 16)]
                        )

            _copy(buf, sem, r).start()
        return 0

    lax.fori_loop(0, RPW // 2, _pair, 0)

    _copy(buf_a, sem_a, RPW - 2).wait()
    _copy(buf_b, sem_b, RPW - 1).wait()


def kernel(x, emb_0, emb_1):
    del x
    e0f = emb_0.reshape(N0 * EMB)
    e1f = emb_1.reshape(N1 * EMB)
    mesh = plsc.VectorSubcoreMesh(core_axis_name="c", subcore_axis_name="s")
    kfn = pl.kernel(
        _sc_body,
        mesh=mesh,
        out_type=jax.ShapeDtypeStruct((N0 * ROW,), jnp.float32),
        scratch_types=[
            pltpu.VMEM((ROW,), jnp.float32),
            pltpu.VMEM((RPW * EMB,), jnp.float32),
            pltpu.VMEM((ROW,), jnp.float32),
            pltpu.VMEM((ROW,), jnp.float32),
            pltpu.SemaphoreType.DMA,
            pltpu.SemaphoreType.DMA,
        ],
        compiler_params=pltpu.CompilerParams(needs_layout_passes=False),
    )
    out_phy = kfn(e0f, e1f)
    out3 = (
        out_phy.reshape(N0, TK, TJ, 8, 128)
        .transpose(0, 1, 3, 2, 4)
        .reshape(N0, EMB, N1)
    )
    return out3.transpose(0, 2, 1)[None]
